# baseline (speedup 1.0000x reference)
import jax
import jax.numpy as jnp
from jax.experimental import pallas as pl
from jax.experimental.pallas import tpu as pltpu


_NC = 8


def _make_body(B, S, D, nc):
    rows = S // nc

    def body(emb_hbm, out_hbm, vmem, read_sems, write_sems):
        reads = [
            pltpu.make_async_copy(
                emb_hbm.at[pl.ds(c * rows, rows), :],
                vmem.at[pl.ds(c * rows, rows), :],
                read_sems.at[c],
            )
            for c in range(nc)
        ]
        for r in reads:
            r.start()
        writes = []
        for c in range(nc):
            reads[c].wait()
            for b in range(B):
                w = pltpu.make_async_copy(
                    vmem.at[pl.ds(c * rows, rows), :],
                    out_hbm.at[b, pl.ds(c * rows, rows), :],
                    write_sems.at[c, b],
                )
                w.start()
                writes.append(w)
        for w in writes:
            w.wait()

    return body


def kernel(h, pos_emb):
    B, S, D = h.shape
    nc = _NC if S % _NC == 0 else 1
    return pl.pallas_call(
        _make_body(B, S, D, nc),
        in_specs=[pl.BlockSpec(memory_space=pl.ANY)],
        out_specs=pl.BlockSpec(memory_space=pl.ANY),
        out_shape=jax.ShapeDtypeStruct((B, S, D), pos_emb.dtype),
        scratch_shapes=[
            pltpu.VMEM((S, D), pos_emb.dtype),
            pltpu.SemaphoreType.DMA((nc,)),
            pltpu.SemaphoreType.DMA((nc, B)),
        ],
    )(pos_emb)


# speedup vs baseline: 1.0336x; 1.0336x over previous
import jax
import jax.numpy as jnp
from jax.experimental import pallas as pl
from jax.experimental.pallas import tpu as pltpu


_NC = 2


def _make_body(B, S, D, nc):
    rows = S // nc

    def body(emb_hbm, out_hbm, vmem, read_sems, write_sems):
        reads = [
            pltpu.make_async_copy(
                emb_hbm.at[pl.ds(c * rows, rows), :],
                vmem.at[pl.ds(c * rows, rows), :],
                read_sems.at[c],
            )
            for c in range(nc)
        ]
        for r in reads:
            r.start()
        writes = []
        for c in range(nc):
            reads[c].wait()
            for b in range(B):
                w = pltpu.make_async_copy(
                    vmem.at[pl.ds(c * rows, rows), :],
                    out_hbm.at[b, pl.ds(c * rows, rows), :],
                    write_sems.at[c, b],
                )
                w.start()
                writes.append(w)
        for w in writes:
            w.wait()

    return body


def kernel(h, pos_emb):
    B, S, D = h.shape
    nc = _NC if S % _NC == 0 else 1
    return pl.pallas_call(
        _make_body(B, S, D, nc),
        in_specs=[pl.BlockSpec(memory_space=pl.ANY)],
        out_specs=pl.BlockSpec(memory_space=pl.ANY),
        out_shape=jax.ShapeDtypeStruct((B, S, D), pos_emb.dtype),
        scratch_shapes=[
            pltpu.VMEM((S, D), pos_emb.dtype),
            pltpu.SemaphoreType.DMA((nc,)),
            pltpu.SemaphoreType.DMA((nc, B)),
        ],
    )(pos_emb)
